# Initial kernel scaffold; baseline (speedup 1.0000x reference)
import jax
import jax.numpy as jnp
from jax.experimental import pallas as pl


def kernel(x, table):
    raise NotImplementedError("write your pallas kernel here")


# speedup vs baseline: 10.2983x; 10.2983x over previous
import functools

import jax
import jax.numpy as jnp
from jax import lax
from jax.experimental import pallas as pl
from jax.experimental.pallas import tpu as pltpu
from jax.experimental.pallas import tpu_sc as plsc

_VOCAB = 100000
_D = 64
_B = 4096
_H = 50

_NC = 2
_NS = 16
_L = 16
_NW = _NC * _NS
_BPW = _B // _NW

_mesh = plsc.VectorSubcoreMesh(
    core_axis_name="c", subcore_axis_name="s", num_cores=_NC, num_subcores=_NS
)


@functools.partial(
    pl.kernel,
    out_type=jax.ShapeDtypeStruct((_B, _D), jnp.float32),
    mesh=_mesh,
    scratch_types=[
        pltpu.VMEM((_H, _BPW), jnp.int32),
        pltpu.VMEM((_BPW, _D), jnp.float32),
        pltpu.SemaphoreType.DMA,
    ],
    compiler_params=pltpu.CompilerParams(use_tc_tiling_on_sc=False),
)
def _cbow_sc(xt_hbm, table_hbm, out_hbm, idx_v, acc_v, sem):
    wid = lax.axis_index("s") * _NC + lax.axis_index("c")
    base = wid * _BPW

    pltpu.sync_copy(xt_hbm.at[:, pl.ds(base, _BPW)], idx_v)

    zeros = jnp.zeros((_L,), jnp.float32)

    @pl.loop(0, _BPW)
    def _zero(r):
        for c in range(_D // _L):
            acc_v[r, pl.ds(c * _L, _L)] = zeros

    @pl.loop(0, _H)
    def _fire(t):
        pltpu.async_copy(table_hbm.at[idx_v.at[t]], acc_v, sem, add=True)

    @pl.loop(0, _H)
    def _drain(t):
        pltpu.make_async_copy(table_hbm.at[idx_v.at[0]], acc_v, sem).wait()

    inv_h = jnp.float32(1.0 / _H)

    @pl.loop(0, _BPW)
    def _scale(r):
        for c in range(_D // _L):
            sl = pl.ds(c * _L, _L)
            acc_v[r, sl] = acc_v[r, sl] * inv_h

    pltpu.sync_copy(acc_v, out_hbm.at[pl.ds(base, _BPW)])


def kernel(x, table):
    xt = x.astype(jnp.int32).T
    return _cbow_sc(xt, table)


# speedup vs baseline: 10.3233x; 1.0024x over previous
import functools

import jax
import jax.numpy as jnp
from jax import lax
from jax.experimental import pallas as pl
from jax.experimental.pallas import tpu as pltpu
from jax.experimental.pallas import tpu_sc as plsc

_VOCAB = 100000
_D = 64
_B = 4096
_H = 50

_NC = 2
_NS = 16
_L = 16
_NW = _NC * _NS
_BPW = _B // _NW

_mesh = plsc.VectorSubcoreMesh(
    core_axis_name="c", subcore_axis_name="s", num_cores=_NC, num_subcores=_NS
)


@functools.partial(
    pl.kernel,
    out_type=jax.ShapeDtypeStruct((_B, _D), jnp.float32),
    mesh=_mesh,
    scratch_types=[
        pltpu.VMEM((_BPW, _H), jnp.int32),
        pltpu.VMEM((_H, _BPW), jnp.int32),
        pltpu.VMEM((_BPW, _D), jnp.float32),
        pltpu.SemaphoreType.DMA,
    ],
    compiler_params=pltpu.CompilerParams(
        use_tc_tiling_on_sc=False, needs_layout_passes=False
    ),
)
def _cbow_sc(x_hbm, table_hbm, out_hbm, raw_v, idxt_v, acc_v, sem):
    wid = lax.axis_index("s") * _NC + lax.axis_index("c")
    base = wid * _BPW

    pltpu.sync_copy(x_hbm.at[pl.ds(base, _BPW)], raw_v)

    zeros = jnp.zeros((_L,), jnp.float32)

    @pl.loop(0, _BPW)
    def _zero(r):
        for c in range(_D // _L):
            acc_v[r, pl.ds(c * _L, _L)] = zeros

    rows0 = lax.iota(jnp.int32, 16)

    @pl.loop(0, _H)
    def _fire(t):
        tv = jnp.zeros((_L,), jnp.int32) + t
        for g in range(_BPW // _L):
            rows = rows0 + g * _L
            idxt_v[t, pl.ds(g * _L, _L)] = plsc.load_gather(raw_v, [rows, tv])
        pltpu.async_copy(table_hbm.at[idxt_v.at[t]], acc_v, sem, add=True)

    @pl.loop(0, _H)
    def _drain(t):
        pltpu.make_async_copy(table_hbm.at[idxt_v.at[0]], acc_v, sem).wait()

    inv_h = jnp.float32(1.0 / _H)

    @pl.loop(0, _BPW)
    def _scale(r):
        for c in range(_D // _L):
            sl = pl.ds(c * _L, _L)
            acc_v[r, sl] = acc_v[r, sl] * inv_h

    pltpu.sync_copy(acc_v, out_hbm.at[pl.ds(base, _BPW)])


def kernel(x, table):
    return _cbow_sc(x.astype(jnp.int32), table)
